# baseline (speedup 1.0000x reference)
import functools

import jax
import jax.numpy as jnp
from jax import lax
from jax.experimental import pallas as pl
from jax.experimental.pallas import tpu as pltpu
from jax.experimental.pallas import tpu_sc as plsc

N = 16384
NC, NS, L = 2, 16, 16
NW = NC * NS
CHUNK = N // NW
STEPS = CHUNK // L


@functools.partial(
    pl.kernel,
    mesh=plsc.VectorSubcoreMesh(core_axis_name="c", subcore_axis_name="s"),
    out_type=jax.ShapeDtypeStruct((N,), jnp.float32),
    compiler_params=pltpu.CompilerParams(
        needs_layout_passes=False, use_tc_tiling_on_sc=False),
    scratch_types=[
        pltpu.VMEM((CHUNK, 2), jnp.float32),
        pltpu.VMEM((CHUNK, 2), jnp.float32),
        pltpu.VMEM((8, 8), jnp.float32),
        pltpu.VMEM((8,), jnp.float32),
        pltpu.VMEM((CHUNK,), jnp.float32),
        pltpu.SemaphoreType.DMA,
    ],
)
def _sc_kernel(x_hbm, y_hbm, alphas_hbm, beta_hbm, out_hbm,
               x_v, y_v, a_v, b_v, o_v, sem):
    wid = lax.axis_index("s") * NC + lax.axis_index("c")
    base = wid * CHUNK
    copies = [
        pltpu.make_async_copy(x_hbm.at[pl.ds(base, CHUNK)], x_v, sem),
        pltpu.make_async_copy(y_hbm.at[pl.ds(base, CHUNK)], y_v, sem),
        pltpu.make_async_copy(alphas_hbm, a_v, sem),
        pltpu.make_async_copy(beta_hbm, b_v, sem),
    ]
    for c in copies:
        c.start()
    for c in copies:
        c.wait()

    lane = lax.iota(jnp.int32, L)
    col0 = jnp.zeros((L,), jnp.int32)
    col1 = col0 + 1
    for j in range(STEPS):
        row = lane + (L * j)
        t_x = plsc.load_gather(x_v, [row, col0])
        t_y = plsc.load_gather(y_v, [row, col0])
        xi = plsc.load_gather(x_v, [row, col1]).astype(jnp.int32)
        yi = plsc.load_gather(y_v, [row, col1]).astype(jnp.int32)
        a = plsc.load_gather(a_v, [xi, yi])
        b = plsc.load_gather(b_v, [yi])
        mask = t_x > 0.0
        tds = jnp.where(mask, t_x - t_y, 0.0)
        o_v[pl.ds(j * L, L)] = jnp.where(mask, a * b * jnp.exp(-b * tds), 0.0)

    pltpu.sync_copy(o_v, out_hbm.at[pl.ds(base, CHUNK)])


def kernel(x, y, alphas, beta):
    return _sc_kernel(x, y, alphas, beta)


# speedup vs baseline: 2.5128x; 2.5128x over previous
import functools

import jax
import jax.numpy as jnp
from jax import lax
from jax.experimental import pallas as pl
from jax.experimental.pallas import tpu as pltpu
from jax.experimental.pallas import tpu_sc as plsc

N = 16384
NC, NS, L = 2, 16, 16
NW = NC * NS
CHUNK = N // NW
BLK = 128
NBLK = CHUNK // BLK


@functools.partial(
    pl.kernel,
    mesh=plsc.VectorSubcoreMesh(core_axis_name="c", subcore_axis_name="s"),
    out_type=jax.ShapeDtypeStruct((N,), jnp.float32),
    compiler_params=pltpu.CompilerParams(needs_layout_passes=False),
    scratch_types=[
        pltpu.VMEM((NBLK, 2, BLK), jnp.float32),
        pltpu.VMEM((NBLK, 2, BLK), jnp.float32),
        pltpu.VMEM((8, 8), jnp.float32),
        pltpu.VMEM((8,), jnp.float32),
        pltpu.VMEM((CHUNK,), jnp.float32),
        pltpu.SemaphoreType.DMA,
    ],
)
def _sc_kernel(x_hbm, y_hbm, alphas_hbm, beta_hbm, out_hbm,
               x_v, y_v, a_v, b_v, o_v, sem):
    wid = lax.axis_index("s") * NC + lax.axis_index("c")
    base = wid * NBLK
    copies = [
        pltpu.make_async_copy(x_hbm.at[pl.ds(base, NBLK)], x_v, sem),
        pltpu.make_async_copy(y_hbm.at[pl.ds(base, NBLK)], y_v, sem),
        pltpu.make_async_copy(alphas_hbm, a_v, sem),
        pltpu.make_async_copy(beta_hbm, b_v, sem),
    ]
    for c in copies:
        c.start()
    for c in copies:
        c.wait()

    for blk in range(NBLK):
        for k in range(BLK // L):
            s = pl.ds(k * L, L)
            t_x = x_v[blk, 0, s]
            t_y = y_v[blk, 0, s]
            xi = x_v[blk, 1, s].astype(jnp.int32)
            yi = y_v[blk, 1, s].astype(jnp.int32)
            a = plsc.load_gather(a_v, [xi, yi])
            b = plsc.load_gather(b_v, [yi])
            mask = t_x > 0.0
            tds = jnp.where(mask, t_x - t_y, 0.0)
            o_v[pl.ds(blk * BLK + k * L, L)] = jnp.where(
                mask, a * b * jnp.exp(-b * tds), 0.0)

    pltpu.sync_copy(o_v, out_hbm.at[pl.ds(wid * CHUNK, CHUNK)])


def kernel(x, y, alphas, beta):
    xb = jnp.swapaxes(x.reshape(BLK, BLK, 2), 1, 2)
    yb = jnp.swapaxes(y.reshape(BLK, BLK, 2), 1, 2)
    return _sc_kernel(xb, yb, alphas, beta)


# speedup vs baseline: 2.6209x; 1.0430x over previous
import functools

import jax
import jax.numpy as jnp
from jax import lax
from jax.experimental import pallas as pl
from jax.experimental.pallas import tpu as pltpu
from jax.experimental.pallas import tpu_sc as plsc

N = 16384
NC, NS, L = 2, 16, 16
NW = NC * NS
CHUNK = N // NW
BLK = 128
NBLK = CHUNK // BLK


@functools.partial(
    pl.kernel,
    mesh=plsc.VectorSubcoreMesh(core_axis_name="c", subcore_axis_name="s"),
    out_type=jax.ShapeDtypeStruct((N,), jnp.float32),
    compiler_params=pltpu.CompilerParams(needs_layout_passes=False),
    scratch_types=[
        pltpu.VMEM((NBLK, 2, BLK), jnp.float32),
        pltpu.VMEM((NBLK, 2, BLK), jnp.float32),
        pltpu.VMEM((8, 8), jnp.float32),
        pltpu.VMEM((8,), jnp.float32),
        pltpu.VMEM((CHUNK,), jnp.float32),
        pltpu.SemaphoreType.DMA,
    ],
)
def _sc_kernel(x_hbm, y_hbm, alphas_hbm, beta_hbm, out_hbm,
               x_v, y_v, a_v, b_v, o_v, sem):
    wid = lax.axis_index("s") * NC + lax.axis_index("c")
    base = wid * NBLK
    copies = [
        pltpu.make_async_copy(x_hbm.at[pl.ds(base, NBLK)], x_v, sem),
        pltpu.make_async_copy(y_hbm.at[pl.ds(base, NBLK)], y_v, sem),
        pltpu.make_async_copy(alphas_hbm, a_v, sem),
        pltpu.make_async_copy(beta_hbm, b_v, sem),
    ]
    for c in copies:
        c.start()
    for c in copies:
        c.wait()

    def step(j, _):
        blk = j >> 3
        k = (j & 7) * L
        s = pl.ds(k, L)
        t_x = x_v[blk, 0, s]
        t_y = y_v[blk, 0, s]
        xi = x_v[blk, 1, s].astype(jnp.int32)
        yi = y_v[blk, 1, s].astype(jnp.int32)
        a = plsc.load_gather(a_v, [xi, yi])
        b = plsc.load_gather(b_v, [yi])
        mask = t_x > 0.0
        tds = jnp.where(mask, t_x - t_y, 0.0)
        o_v[pl.ds(blk * BLK + k, L)] = jnp.where(
            mask, a * b * jnp.exp(-b * tds), 0.0)
        return _

    lax.fori_loop(0, CHUNK // L, step, None)

    pltpu.sync_copy(o_v, out_hbm.at[pl.ds(wid * CHUNK, CHUNK)])


def kernel(x, y, alphas, beta):
    xb = jnp.swapaxes(x.reshape(BLK, BLK, 2), 1, 2)
    yb = jnp.swapaxes(y.reshape(BLK, BLK, 2), 1, 2)
    return _sc_kernel(xb, yb, alphas, beta)


# speedup vs baseline: 2.6561x; 1.0134x over previous
import functools

import jax
import jax.numpy as jnp
from jax import lax
from jax.experimental import pallas as pl
from jax.experimental.pallas import tpu as pltpu
from jax.experimental.pallas import tpu_sc as plsc

N = 16384
NC, NS, L = 2, 16, 16
NW = NC * NS
CHUNK = N // NW
BLK = 128
NBLK = CHUNK // BLK


@functools.partial(
    pl.kernel,
    mesh=plsc.VectorSubcoreMesh(core_axis_name="c", subcore_axis_name="s"),
    out_type=jax.ShapeDtypeStruct((N,), jnp.float32),
    compiler_params=pltpu.CompilerParams(needs_layout_passes=False),
    scratch_types=[
        pltpu.VMEM((NBLK, 2, BLK), jnp.float32),
        pltpu.VMEM((NBLK, 2, BLK), jnp.float32),
        pltpu.VMEM((8, 8), jnp.float32),
        pltpu.VMEM((8,), jnp.float32),
        pltpu.VMEM((CHUNK,), jnp.float32),
        pltpu.SemaphoreType.DMA,
    ],
)
def _sc_kernel(x_hbm, y_hbm, alphas_hbm, beta_hbm, out_hbm,
               x_v, y_v, a_v, b_v, o_v, sem):
    wid = lax.axis_index("s") * NC + lax.axis_index("c")
    base = wid * NBLK
    copies = [
        pltpu.make_async_copy(x_hbm.at[pl.ds(base, NBLK)], x_v, sem),
        pltpu.make_async_copy(y_hbm.at[pl.ds(base, NBLK)], y_v, sem),
        pltpu.make_async_copy(alphas_hbm, a_v, sem),
        pltpu.make_async_copy(beta_hbm, b_v, sem),
    ]
    for c in copies:
        c.start()
    for c in copies:
        c.wait()

    def step(j):
        blk = j >> 3
        k = (j & 7) * L
        s = pl.ds(k, L)
        t_x = x_v[blk, 0, s]
        t_y = y_v[blk, 0, s]
        xi = x_v[blk, 1, s].astype(jnp.int32)
        yi = y_v[blk, 1, s].astype(jnp.int32)
        a = plsc.load_gather(a_v, [xi, yi])
        b = plsc.load_gather(b_v, [yi])
        o_v[pl.ds(blk * BLK + k, L)] = jnp.where(
            t_x > 0.0, a * b * jnp.exp(b * (t_y - t_x)), 0.0)

    plsc.parallel_loop(0, CHUNK // L, 1, unroll=4)(step)

    pltpu.sync_copy(o_v, out_hbm.at[pl.ds(wid * CHUNK, CHUNK)])


def kernel(x, y, alphas, beta):
    xb = jnp.swapaxes(x.reshape(BLK, BLK, 2), 1, 2)
    yb = jnp.swapaxes(y.reshape(BLK, BLK, 2), 1, 2)
    return _sc_kernel(xb, yb, alphas, beta)
